# baseline (speedup 1.0000x reference)
import functools

import jax
import jax.numpy as jnp
from jax import lax
from jax.experimental import pallas as pl
from jax.experimental.pallas import tpu as pltpu
from jax.experimental.pallas import tpu_sc as plsc

NUM_ROWS = 17
D = 128
LANES = 16
CHUNK = 320


@functools.lru_cache(maxsize=None)
def _build(batch: int):
    info = plsc.get_sparse_core_info()
    nw = info.num_cores * info.num_subcores
    assert batch % (nw * 2 * CHUNK) == 0
    b_per_w = batch // nw
    n_pairs = b_per_w // (2 * CHUNK)
    mesh = plsc.VectorSubcoreMesh(core_axis_name="c", subcore_axis_name="s")

    @functools.partial(
        pl.kernel,
        out_type=jax.ShapeDtypeStruct((batch, D), jnp.float32),
        mesh=mesh,
        compiler_params=pltpu.CompilerParams(needs_layout_passes=False),
        scratch_types=[
            pltpu.VMEM((NUM_ROWS, D), jnp.float32),
            pltpu.VMEM((b_per_w,), jnp.int32),
            pltpu.VMEM((CHUNK, D), jnp.float32),
            pltpu.VMEM((CHUNK, D), jnp.float32),
            pltpu.SemaphoreType.DMA,
            pltpu.SemaphoreType.DMA,
        ],
    )
    def gather_kernel(idx_hbm, table_hbm, out_hbm, table_v, idx_all,
                      rows_a, rows_b, sem_a, sem_b):
        wid = lax.axis_index("s") * info.num_cores + lax.axis_index("c")
        base = wid * b_per_w
        pltpu.sync_copy(table_hbm, table_v)
        pltpu.sync_copy(idx_hbm.at[pl.ds(base, b_per_w)], idx_all)

        lane_iota = lax.iota(jnp.int32, LANES)

        def do_chunk(g, rows_v):
            off = g * CHUNK

            def group_body(t, carry):
                vidx = idx_all[pl.ds(off + t * LANES, LANES)]
                row_vec = lane_iota + t * LANES
                for c in range(D):
                    col = jnp.full((LANES,), c, jnp.int32)
                    vals = plsc.load_gather(table_v, [vidx, col])
                    plsc.store_scatter(rows_v, [row_vec, col], vals)
                return carry

            lax.fori_loop(0, CHUNK // LANES, group_body, 0)

        def pair_body(gg, carry):
            for b, rows_v, sem in ((0, rows_a, sem_a), (1, rows_b, sem_b)):
                g = gg * 2 + b

                @pl.when(gg > 0)
                def _wait():
                    pltpu.make_async_copy(
                        rows_v, out_hbm.at[pl.ds(base, CHUNK)], sem).wait()

                do_chunk(g, rows_v)
                pltpu.async_copy(
                    rows_v, out_hbm.at[pl.ds(base + g * CHUNK, CHUNK)], sem)
            return carry

        lax.fori_loop(0, n_pairs, pair_body, 0)
        for rows_v, sem in ((rows_a, sem_a), (rows_b, sem_b)):
            pltpu.make_async_copy(
                rows_v, out_hbm.at[pl.ds(base, CHUNK)], sem).wait()

    return gather_kernel


def kernel(hierarchy_labels, weight):
    b, l = hierarchy_labels.shape
    idx = hierarchy_labels.reshape(-1).astype(jnp.int32)
    out = _build(b * l)(idx, weight)
    return out.reshape(b, l, D)


# speedup vs baseline: 18.8234x; 18.8234x over previous
import functools

import jax
import jax.numpy as jnp
from jax import lax
from jax.experimental import pallas as pl
from jax.experimental.pallas import tpu as pltpu
from jax.experimental.pallas import tpu_sc as plsc

NUM_ROWS = 17
D = 128
CHUNK = 128


@functools.lru_cache(maxsize=None)
def _build(batch: int):
    info = plsc.get_sparse_core_info()
    nw = info.num_cores * info.num_subcores
    assert batch % (nw * 2 * CHUNK) == 0
    b_per_w = batch // nw
    n_chunks = b_per_w // CHUNK
    mesh = plsc.VectorSubcoreMesh(core_axis_name="c", subcore_axis_name="s")

    @functools.partial(
        pl.kernel,
        out_type=jax.ShapeDtypeStruct((batch, D), jnp.float32),
        mesh=mesh,
        scratch_types=[
            pltpu.VMEM_SHARED((NUM_ROWS, D), jnp.float32),
            pltpu.VMEM((n_chunks, CHUNK), jnp.int32),
            pltpu.VMEM((CHUNK, D), jnp.float32),
            pltpu.VMEM((CHUNK, D), jnp.float32),
            pltpu.SemaphoreType.DMA,
            pltpu.SemaphoreType.DMA,
            pltpu.SemaphoreType.DMA,
        ],
    )
    def gather_kernel(idx_hbm, table_hbm, out_hbm, table_sh, idx_all,
                      rows_a, rows_b, gsem, osem_a, osem_b):
        wid = lax.axis_index("s") * info.num_cores + lax.axis_index("c")
        base = wid * b_per_w

        @pl.when(lax.axis_index("s") == 0)
        def _stage_table():
            pltpu.sync_copy(table_hbm, table_sh)

        pltpu.sync_copy(
            idx_hbm.at[pl.ds(wid * n_chunks, n_chunks)], idx_all)
        plsc.subcore_barrier()

        def pair_body(gg, carry):
            for b, rows_v, osem in ((0, rows_a, osem_a), (1, rows_b, osem_b)):
                g = gg * 2 + b

                @pl.when(gg > 0)
                def _wait_out():
                    pltpu.make_async_copy(
                        rows_v, out_hbm.at[pl.ds(base, CHUNK)], osem).wait()

                pltpu.async_copy(
                    table_sh.at[idx_all.at[g]], rows_v, gsem).wait()
                pltpu.async_copy(
                    rows_v, out_hbm.at[pl.ds(base + g * CHUNK, CHUNK)], osem)
            return carry

        lax.fori_loop(0, n_chunks // 2, pair_body, 0)
        for rows_v, osem in ((rows_a, osem_a), (rows_b, osem_b)):
            pltpu.make_async_copy(
                rows_v, out_hbm.at[pl.ds(base, CHUNK)], osem).wait()

    return gather_kernel


def kernel(hierarchy_labels, weight):
    b, l = hierarchy_labels.shape
    idx = hierarchy_labels.reshape(-1, CHUNK).astype(jnp.int32)
    out = _build(b * l)(idx, weight)
    return out.reshape(b, l, D)


# speedup vs baseline: 20.2307x; 1.0748x over previous
import functools

import jax
import jax.numpy as jnp
from jax import lax
from jax.experimental import pallas as pl
from jax.experimental.pallas import tpu as pltpu
from jax.experimental.pallas import tpu_sc as plsc

NUM_ROWS = 17
D = 128
CHUNK = 128
NBUF = 4


@functools.lru_cache(maxsize=None)
def _build(batch: int):
    info = plsc.get_sparse_core_info()
    nw = info.num_cores * info.num_subcores
    assert batch % (nw * NBUF * CHUNK) == 0
    b_per_w = batch // nw
    n_chunks = b_per_w // CHUNK
    mesh = plsc.VectorSubcoreMesh(core_axis_name="c", subcore_axis_name="s")

    @functools.partial(
        pl.kernel,
        out_type=jax.ShapeDtypeStruct((batch, D), jnp.float32),
        mesh=mesh,
        scratch_types=[
            pltpu.VMEM_SHARED((NUM_ROWS, D), jnp.float32),
            pltpu.VMEM((n_chunks, CHUNK), jnp.int32),
        ]
        + [pltpu.VMEM((CHUNK, D), jnp.float32)] * NBUF
        + [pltpu.SemaphoreType.DMA] * (2 * NBUF),
    )
    def gather_kernel(idx_hbm, table_hbm, out_hbm, table_sh, idx_all, *bufs):
        rows = bufs[:NBUF]
        gsem = bufs[NBUF:2 * NBUF]
        osem = bufs[2 * NBUF:]
        wid = lax.axis_index("s") * info.num_cores + lax.axis_index("c")
        base = wid * b_per_w

        @pl.when(lax.axis_index("s") == 0)
        def _stage_table():
            pltpu.sync_copy(table_hbm, table_sh)

        pltpu.sync_copy(
            idx_hbm.at[pl.ds(wid * n_chunks, n_chunks)], idx_all)
        plsc.subcore_barrier()

        def start_gather(g, b):
            pltpu.async_copy(table_sh.at[idx_all.at[g]], rows[b], gsem[b])

        def wait_gather(b):
            pltpu.make_async_copy(
                table_sh.at[idx_all.at[0]], rows[b], gsem[b]).wait()

        def start_out(g, b):
            pltpu.async_copy(
                rows[b], out_hbm.at[pl.ds(base + g * CHUNK, CHUNK)], osem[b])

        def wait_out(b):
            pltpu.make_async_copy(
                rows[b], out_hbm.at[pl.ds(base, CHUNK)], osem[b]).wait()

        start_gather(0, 0)
        start_gather(1, 1)

        def ring_body(gg, carry):
            for p in range(NBUF):
                g = gg * NBUF + p
                b = p
                b2 = (p + 2) % NBUF

                @pl.when(g >= 2)
                def _drain_out():
                    wait_out(b2)

                @pl.when(g + 2 < n_chunks)
                def _prefetch():
                    start_gather(g + 2, b2)

                wait_gather(b)
                start_out(g, b)
            return carry

        lax.fori_loop(0, n_chunks // NBUF, ring_body, 0)
        wait_out((n_chunks - 2) % NBUF)
        wait_out((n_chunks - 1) % NBUF)

    return gather_kernel


def kernel(hierarchy_labels, weight):
    b, l = hierarchy_labels.shape
    idx = hierarchy_labels.reshape(-1, CHUNK).astype(jnp.int32)
    out = _build(b * l)(idx, weight)
    return out.reshape(b, l, D)
